# baseline (speedup 1.0000x reference)
import jax
import jax.numpy as jnp
from jax.experimental import pallas as pl
from jax.experimental.pallas import tpu as pltpu

_B = 16384
_D = 256
_CHUNK = 64
_T = _B // _CHUNK


def _body(pe_ref, out_hbm, buf, sem):
    buf[...] = jnp.broadcast_to(pe_ref[...], buf.shape)
    copies = [
        pltpu.make_async_copy(buf, out_hbm.at[pl.ds(t * _CHUNK, _CHUNK), 0, :], sem)
        for t in range(_T)
    ]
    for c in copies:
        c.start()
    for c in copies:
        c.wait()


def kernel(batch_size, pos_embed):
    return pl.pallas_call(
        _body,
        in_specs=[pl.BlockSpec(memory_space=pltpu.VMEM)],
        out_specs=pl.BlockSpec(memory_space=pltpu.HBM),
        out_shape=jax.ShapeDtypeStruct((_B, 1, _D), jnp.float32),
        scratch_shapes=[
            pltpu.VMEM((_CHUNK, _D), jnp.float32),
            pltpu.SemaphoreType.DMA,
        ],
    )(pos_embed)


# speedup vs baseline: 1.0512x; 1.0512x over previous
import jax
import jax.numpy as jnp
from jax.experimental import pallas as pl
from jax.experimental.pallas import tpu as pltpu

_B = 16384
_D = 256
_CHUNK = 128
_T = _B // _CHUNK


def _body(pe_ref, out_hbm, buf, sem):
    buf[...] = jnp.broadcast_to(pe_ref[...], buf.shape)
    copies = [
        pltpu.make_async_copy(buf, out_hbm.at[pl.ds(t * _CHUNK, _CHUNK), 0, :], sem)
        for t in range(_T)
    ]
    for c in copies:
        c.start()
    for c in copies:
        c.wait()


def kernel(batch_size, pos_embed):
    return pl.pallas_call(
        _body,
        in_specs=[pl.BlockSpec(memory_space=pltpu.VMEM)],
        out_specs=pl.BlockSpec(memory_space=pltpu.HBM),
        out_shape=jax.ShapeDtypeStruct((_B, 1, _D), jnp.float32),
        scratch_shapes=[
            pltpu.VMEM((_CHUNK, _D), jnp.float32),
            pltpu.SemaphoreType.DMA,
        ],
    )(pos_embed)
